# Initial kernel scaffold; baseline (speedup 1.0000x reference)
import jax
import jax.numpy as jnp
from jax.experimental import pallas as pl


def kernel(h, batch, W0, b0, W1, b1, energy_scaling_coeff, energy_shifting_coeff):
    raise NotImplementedError("write your pallas kernel here")


# speedup vs baseline: 1.4026x; 1.4026x over previous
import functools

import jax
import jax.numpy as jnp
from jax import lax
from jax.experimental import pallas as pl
from jax.experimental.pallas import tpu as pltpu
from jax.experimental.pallas import tpu_sc as plsc

N = 100000
D = 128
S = 1024

NTILES = 16
JROWS = 49
CHUNK = JROWS * 128
NPAD = NTILES * CHUNK
SEG_T = S // NTILES

ROWS_BLK = 4000


def _dot_body(h_ref, w0_ref, b0_ref, w1_ref, b1_ref, sc_ref, o_ref):
    wc = jnp.dot(
        w0_ref[...], w1_ref[...],
        preferred_element_type=jnp.float32,
        precision=lax.Precision.HIGHEST,
    )
    cc = (
        jnp.dot(
            b0_ref[...], w1_ref[...],
            preferred_element_type=jnp.float32,
            precision=lax.Precision.HIGHEST,
        )
        + b1_ref[...]
    )
    s = sc_ref[0, 0]
    v = jnp.dot(
        h_ref[...], wc,
        preferred_element_type=jnp.float32,
        precision=lax.Precision.HIGHEST,
    )
    o_ref[...] = (v + cc) * s


def _node_scalars(h, W0, b0, W1, b1, scale):
    grid = N // ROWS_BLK
    return pl.pallas_call(
        _dot_body,
        grid=(grid,),
        in_specs=[
            pl.BlockSpec((ROWS_BLK, D), lambda i: (i, 0)),
            pl.BlockSpec((D, 64), lambda i: (0, 0)),
            pl.BlockSpec((1, 64), lambda i: (0, 0)),
            pl.BlockSpec((64, 1), lambda i: (0, 0)),
            pl.BlockSpec((1, 1), lambda i: (0, 0)),
            pl.BlockSpec((1, 1), lambda i: (0, 0)),
        ],
        out_specs=pl.BlockSpec((ROWS_BLK, 1), lambda i: (i, 0)),
        out_shape=jax.ShapeDtypeStruct((N, 1), jnp.float32),
    )(h, W0, b0.reshape(1, 64), W1, b1.reshape(1, 1), scale.reshape(1, 1))


def _segsum_body(
    v_hbm, b_hbm, shift_hbm, out_hbm,
    vals_v, idx_v, acc_v, red_v, shi_v, out_v, acc_sh, sem,
):
    sid = lax.axis_index("s")
    base = sid * CHUNK

    zero16 = jnp.zeros((16,), jnp.float32)
    for c in range(S // 16):
        acc_v[pl.ds(c * 16, 16)] = zero16
    pltpu.sync_copy(acc_v, acc_sh.at[pl.ds(sid * S, S)])

    pltpu.sync_copy(v_hbm.at[pl.ds(base, CHUNK)], vals_v)
    pltpu.sync_copy(b_hbm.at[sid], idx_v)

    descs = []
    for j in range(JROWS):
        descs.append(
            pltpu.async_copy(
                vals_v.at[pl.ds(j * 128, 128)],
                acc_sh.at[idx_v.at[j]],
                sem,
                add=True,
            )
        )
    for d in descs:
        d.wait()
    plsc.subcore_barrier()

    col = sid * SEG_T
    for j in range(NTILES):
        pltpu.sync_copy(acc_sh.at[pl.ds(j * S + col, SEG_T)], red_v.at[j])
    pltpu.sync_copy(shift_hbm.at[pl.ds(col, SEG_T)], shi_v)
    for c in range(SEG_T // 16):
        acc = shi_v[pl.ds(c * 16, 16)]
        for j in range(NTILES):
            acc = acc + red_v[j, pl.ds(c * 16, 16)]
        out_v[pl.ds(c * 16, 16)] = acc
    pltpu.sync_copy(out_v, out_hbm.at[pl.ds(col, SEG_T)])


@functools.partial(
    pl.kernel,
    out_type=jax.ShapeDtypeStruct((S,), jnp.float32),
    mesh=plsc.VectorSubcoreMesh(
        core_axis_name="c", subcore_axis_name="s", num_cores=1
    ),
    scratch_types=[
        pltpu.VMEM((CHUNK,), jnp.float32),
        pltpu.VMEM((JROWS, 128), jnp.int32),
        pltpu.VMEM((S,), jnp.float32),
        pltpu.VMEM((NTILES, SEG_T), jnp.float32),
        pltpu.VMEM((SEG_T,), jnp.float32),
        pltpu.VMEM((SEG_T,), jnp.float32),
        pltpu.VMEM_SHARED((NTILES * S,), jnp.float32),
        pltpu.SemaphoreType.DMA,
    ],
)
def _segsum(
    v_hbm, b_hbm, shift_hbm, out_hbm,
    vals_v, idx_v, acc_v, red_v, shi_v, out_v, acc_sh, sem,
):
    _segsum_body(
        v_hbm, b_hbm, shift_hbm, out_hbm,
        vals_v, idx_v, acc_v, red_v, shi_v, out_v, acc_sh, sem,
    )


def kernel(h, batch, W0, b0, W1, b1, energy_scaling_coeff, energy_shifting_coeff):
    v = _node_scalars(h, W0, b0, W1, b1, energy_scaling_coeff).reshape(-1)
    pad = NPAD - N
    vp = jnp.concatenate([v, jnp.zeros((pad,), jnp.float32)])
    bp = jnp.concatenate([batch, jnp.zeros((pad,), jnp.int32)])
    bp = bp.reshape(NTILES, JROWS, 128)
    bp = bp + (jnp.arange(NTILES, dtype=jnp.int32) * S)[:, None, None]
    shift = jnp.broadcast_to(
        energy_shifting_coeff.astype(jnp.float32), (S,)
    )
    return _segsum(vp, bp, shift)


# speedup vs baseline: 1.6314x; 1.1631x over previous
import functools

import jax
import jax.numpy as jnp
from jax import lax
from jax.experimental import pallas as pl
from jax.experimental.pallas import tpu as pltpu
from jax.experimental.pallas import tpu_sc as plsc

N = 100000
D = 128
S = 1024

NTILES = 16
JROWS = 49
CHUNK = JROWS * 128
NPAD = NTILES * CHUNK
SEG_T = S // NTILES

ROWS_BLK = 4000


def _dot_body(h_ref, w0_ref, b0_ref, w1_ref, b1_ref, sc_ref, o_ref):
    wc = jnp.dot(
        w0_ref[...], w1_ref[...],
        preferred_element_type=jnp.float32,
        precision=lax.Precision.HIGHEST,
    )
    cc = (
        jnp.dot(
            b0_ref[...], w1_ref[...],
            preferred_element_type=jnp.float32,
            precision=lax.Precision.HIGHEST,
        )
        + b1_ref[...]
    )
    s = sc_ref[0, 0]
    v = jnp.dot(
        h_ref[...], wc,
        preferred_element_type=jnp.float32,
    )
    o_ref[...] = (v + cc) * s


def _node_scalars(h, W0, b0, W1, b1, scale):
    grid = N // ROWS_BLK
    return pl.pallas_call(
        _dot_body,
        grid=(grid,),
        in_specs=[
            pl.BlockSpec((ROWS_BLK, D), lambda i: (i, 0)),
            pl.BlockSpec((D, 64), lambda i: (0, 0)),
            pl.BlockSpec((1, 64), lambda i: (0, 0)),
            pl.BlockSpec((64, 1), lambda i: (0, 0)),
            pl.BlockSpec((1, 1), lambda i: (0, 0)),
            pl.BlockSpec((1, 1), lambda i: (0, 0)),
        ],
        out_specs=pl.BlockSpec((ROWS_BLK, 1), lambda i: (i, 0)),
        out_shape=jax.ShapeDtypeStruct((N, 1), jnp.float32),
    )(h, W0, b0.reshape(1, 64), W1, b1.reshape(1, 1), scale.reshape(1, 1))


def _segsum_body(
    v_hbm, b_hbm, shift_hbm, out_hbm,
    vals_v, idx_v, acc_v, red_v, shi_v, out_v, acc_sh, sem,
):
    sid = lax.axis_index("s")
    base = sid * CHUNK

    zero16 = jnp.zeros((16,), jnp.float32)
    for c in range(S // 16):
        acc_v[pl.ds(c * 16, 16)] = zero16
    pltpu.sync_copy(acc_v, acc_sh.at[pl.ds(sid * S, S)])

    pltpu.sync_copy(v_hbm.at[pl.ds(base, CHUNK)], vals_v)
    pltpu.sync_copy(b_hbm.at[sid], idx_v)

    descs = []
    for j in range(JROWS):
        descs.append(
            pltpu.async_copy(
                vals_v.at[pl.ds(j * 128, 128)],
                acc_sh.at[idx_v.at[j]],
                sem,
                add=True,
            )
        )
    for d in descs:
        d.wait()
    plsc.subcore_barrier()

    col = sid * SEG_T
    for j in range(NTILES):
        pltpu.sync_copy(acc_sh.at[pl.ds(j * S + col, SEG_T)], red_v.at[j])
    pltpu.sync_copy(shift_hbm.at[pl.ds(col, SEG_T)], shi_v)
    for c in range(SEG_T // 16):
        acc = shi_v[pl.ds(c * 16, 16)]
        for j in range(NTILES):
            acc = acc + red_v[j, pl.ds(c * 16, 16)]
        out_v[pl.ds(c * 16, 16)] = acc
    pltpu.sync_copy(out_v, out_hbm.at[pl.ds(col, SEG_T)])


@functools.partial(
    pl.kernel,
    out_type=jax.ShapeDtypeStruct((S,), jnp.float32),
    mesh=plsc.VectorSubcoreMesh(
        core_axis_name="c", subcore_axis_name="s", num_cores=1
    ),
    scratch_types=[
        pltpu.VMEM((CHUNK,), jnp.float32),
        pltpu.VMEM((JROWS, 128), jnp.int32),
        pltpu.VMEM((S,), jnp.float32),
        pltpu.VMEM((NTILES, SEG_T), jnp.float32),
        pltpu.VMEM((SEG_T,), jnp.float32),
        pltpu.VMEM((SEG_T,), jnp.float32),
        pltpu.VMEM_SHARED((NTILES * S,), jnp.float32),
        pltpu.SemaphoreType.DMA,
    ],
)
def _segsum(
    v_hbm, b_hbm, shift_hbm, out_hbm,
    vals_v, idx_v, acc_v, red_v, shi_v, out_v, acc_sh, sem,
):
    _segsum_body(
        v_hbm, b_hbm, shift_hbm, out_hbm,
        vals_v, idx_v, acc_v, red_v, shi_v, out_v, acc_sh, sem,
    )


def kernel(h, batch, W0, b0, W1, b1, energy_scaling_coeff, energy_shifting_coeff):
    v = _node_scalars(h, W0, b0, W1, b1, energy_scaling_coeff).reshape(-1)
    pad = NPAD - N
    vp = jnp.concatenate([v, jnp.zeros((pad,), jnp.float32)])
    bp = jnp.concatenate([batch, jnp.zeros((pad,), jnp.int32)])
    bp = bp.reshape(NTILES, JROWS, 128)
    bp = bp + (jnp.arange(NTILES, dtype=jnp.int32) * S)[:, None, None]
    shift = jnp.broadcast_to(
        energy_shifting_coeff.astype(jnp.float32), (S,)
    )
    return _segsum(vp, bp, shift)


# speedup vs baseline: 1.7775x; 1.0895x over previous
import functools

import jax
import jax.numpy as jnp
from jax import lax
from jax.experimental import pallas as pl
from jax.experimental.pallas import tpu as pltpu
from jax.experimental.pallas import tpu_sc as plsc

N = 100000
D = 128
S = 1024

NTILES = 16
JROWS = 49
CHUNK = JROWS * 128
NPAD = NTILES * CHUNK
SEG_T = S // NTILES

ROWS_BLK = 10000


def _dot_body(h_ref, w0_ref, b0_ref, w1_ref, b1_ref, sc_ref, o_ref):
    wc = jnp.dot(
        w0_ref[...], w1_ref[...],
        preferred_element_type=jnp.float32,
        precision=lax.Precision.HIGHEST,
    )
    cc = (
        jnp.dot(
            b0_ref[...], w1_ref[...],
            preferred_element_type=jnp.float32,
            precision=lax.Precision.HIGHEST,
        )
        + b1_ref[...]
    )
    s = sc_ref[0, 0]
    v = jnp.dot(
        h_ref[...], wc,
        preferred_element_type=jnp.float32,
    )
    o_ref[...] = (v + cc) * s


def _node_scalars(h, W0, b0, W1, b1, scale):
    grid = N // ROWS_BLK
    return pl.pallas_call(
        _dot_body,
        grid=(grid,),
        in_specs=[
            pl.BlockSpec((ROWS_BLK, D), lambda i: (i, 0)),
            pl.BlockSpec((D, 64), lambda i: (0, 0)),
            pl.BlockSpec((1, 64), lambda i: (0, 0)),
            pl.BlockSpec((64, 1), lambda i: (0, 0)),
            pl.BlockSpec((1, 1), lambda i: (0, 0)),
            pl.BlockSpec((1, 1), lambda i: (0, 0)),
        ],
        out_specs=pl.BlockSpec((ROWS_BLK, 1), lambda i: (i, 0)),
        out_shape=jax.ShapeDtypeStruct((N, 1), jnp.float32),
    )(h, W0, b0.reshape(1, 64), W1, b1.reshape(1, 1), scale.reshape(1, 1))


def _segsum_body(
    v_hbm, b_hbm, shift_hbm, out_hbm,
    vals_v, idx_v, acc_v, red_v, shi_v, out_v, acc_sh, sem,
):
    sid = lax.axis_index("s")
    base = sid * CHUNK

    zero16 = jnp.zeros((16,), jnp.float32)
    for c in range(S // 16):
        acc_v[pl.ds(c * 16, 16)] = zero16
    pltpu.sync_copy(acc_v, acc_sh.at[pl.ds(sid * S, S)])

    pltpu.sync_copy(v_hbm.at[pl.ds(base, CHUNK)], vals_v)
    pltpu.sync_copy(b_hbm.at[sid], idx_v)

    descs = []
    for j in range(JROWS):
        descs.append(
            pltpu.async_copy(
                vals_v.at[pl.ds(j * 128, 128)],
                acc_sh.at[idx_v.at[j]],
                sem,
                add=True,
            )
        )
    for d in descs:
        d.wait()
    plsc.subcore_barrier()

    col = sid * SEG_T
    for j in range(NTILES):
        pltpu.sync_copy(acc_sh.at[pl.ds(j * S + col, SEG_T)], red_v.at[j])
    pltpu.sync_copy(shift_hbm.at[pl.ds(col, SEG_T)], shi_v)
    for c in range(SEG_T // 16):
        acc = shi_v[pl.ds(c * 16, 16)]
        for j in range(NTILES):
            acc = acc + red_v[j, pl.ds(c * 16, 16)]
        out_v[pl.ds(c * 16, 16)] = acc
    pltpu.sync_copy(out_v, out_hbm.at[pl.ds(col, SEG_T)])


@functools.partial(
    pl.kernel,
    out_type=jax.ShapeDtypeStruct((S,), jnp.float32),
    mesh=plsc.VectorSubcoreMesh(
        core_axis_name="c", subcore_axis_name="s", num_cores=1
    ),
    scratch_types=[
        pltpu.VMEM((CHUNK,), jnp.float32),
        pltpu.VMEM((JROWS, 128), jnp.int32),
        pltpu.VMEM((S,), jnp.float32),
        pltpu.VMEM((NTILES, SEG_T), jnp.float32),
        pltpu.VMEM((SEG_T,), jnp.float32),
        pltpu.VMEM((SEG_T,), jnp.float32),
        pltpu.VMEM_SHARED((NTILES * S,), jnp.float32),
        pltpu.SemaphoreType.DMA,
    ],
)
def _segsum(
    v_hbm, b_hbm, shift_hbm, out_hbm,
    vals_v, idx_v, acc_v, red_v, shi_v, out_v, acc_sh, sem,
):
    _segsum_body(
        v_hbm, b_hbm, shift_hbm, out_hbm,
        vals_v, idx_v, acc_v, red_v, shi_v, out_v, acc_sh, sem,
    )


def kernel(h, batch, W0, b0, W1, b1, energy_scaling_coeff, energy_shifting_coeff):
    v = _node_scalars(h, W0, b0, W1, b1, energy_scaling_coeff).reshape(-1)
    pad = NPAD - N
    vp = jnp.concatenate([v, jnp.zeros((pad,), jnp.float32)])
    bp = jnp.concatenate([batch, jnp.zeros((pad,), jnp.int32)])
    bp = bp.reshape(NTILES, JROWS, 128)
    bp = bp + (jnp.arange(NTILES, dtype=jnp.int32) * S)[:, None, None]
    shift = jnp.broadcast_to(
        energy_shifting_coeff.astype(jnp.float32), (S,)
    )
    return _segsum(vp, bp, shift)


# speedup vs baseline: 1.8053x; 1.0156x over previous
import functools

import jax
import jax.numpy as jnp
from jax import lax
from jax.experimental import pallas as pl
from jax.experimental.pallas import tpu as pltpu
from jax.experimental.pallas import tpu_sc as plsc

N = 100000
D = 128
S = 1024

NTILES = 16
JROWS = 49
CHUNK = JROWS * 128
NPAD = NTILES * CHUNK
SEG_T = S // NTILES

ROWS_BLK = 20000


def _dot_body(h_ref, w0_ref, b0_ref, w1_ref, b1_ref, sc_ref, o_ref):
    wc = jnp.dot(
        w0_ref[...], w1_ref[...],
        preferred_element_type=jnp.float32,
        precision=lax.Precision.HIGHEST,
    )
    cc = (
        jnp.dot(
            b0_ref[...], w1_ref[...],
            preferred_element_type=jnp.float32,
            precision=lax.Precision.HIGHEST,
        )
        + b1_ref[...]
    )
    s = sc_ref[0, 0]
    v = jnp.dot(
        h_ref[...], wc,
        preferred_element_type=jnp.float32,
    )
    o_ref[...] = (v + cc) * s


def _node_scalars(h, W0, b0, W1, b1, scale):
    grid = N // ROWS_BLK
    return pl.pallas_call(
        _dot_body,
        grid=(grid,),
        in_specs=[
            pl.BlockSpec((ROWS_BLK, D), lambda i: (i, 0)),
            pl.BlockSpec((D, 64), lambda i: (0, 0)),
            pl.BlockSpec((1, 64), lambda i: (0, 0)),
            pl.BlockSpec((64, 1), lambda i: (0, 0)),
            pl.BlockSpec((1, 1), lambda i: (0, 0)),
            pl.BlockSpec((1, 1), lambda i: (0, 0)),
        ],
        out_specs=pl.BlockSpec((ROWS_BLK, 1), lambda i: (i, 0)),
        out_shape=jax.ShapeDtypeStruct((N, 1), jnp.float32),
    )(h, W0, b0.reshape(1, 64), W1, b1.reshape(1, 1), scale.reshape(1, 1))


def _segsum_body(
    v_hbm, b_hbm, shift_hbm, out_hbm,
    vals_v, idx_v, acc_v, red_v, shi_v, out_v, acc_sh, sem,
):
    sid = lax.axis_index("s")
    base = sid * CHUNK

    zero16 = jnp.zeros((16,), jnp.float32)
    for c in range(S // 16):
        acc_v[pl.ds(c * 16, 16)] = zero16
    pltpu.sync_copy(acc_v, acc_sh.at[pl.ds(sid * S, S)])

    pltpu.sync_copy(v_hbm.at[pl.ds(base, CHUNK)], vals_v)
    pltpu.sync_copy(b_hbm.at[sid], idx_v)

    descs = []
    for j in range(JROWS):
        descs.append(
            pltpu.async_copy(
                vals_v.at[pl.ds(j * 128, 128)],
                acc_sh.at[idx_v.at[j]],
                sem,
                add=True,
            )
        )
    for d in descs:
        d.wait()
    plsc.subcore_barrier()

    col = sid * SEG_T
    for j in range(NTILES):
        pltpu.sync_copy(acc_sh.at[pl.ds(j * S + col, SEG_T)], red_v.at[j])
    pltpu.sync_copy(shift_hbm.at[pl.ds(col, SEG_T)], shi_v)
    for c in range(SEG_T // 16):
        acc = shi_v[pl.ds(c * 16, 16)]
        for j in range(NTILES):
            acc = acc + red_v[j, pl.ds(c * 16, 16)]
        out_v[pl.ds(c * 16, 16)] = acc
    pltpu.sync_copy(out_v, out_hbm.at[pl.ds(col, SEG_T)])


@functools.partial(
    pl.kernel,
    out_type=jax.ShapeDtypeStruct((S,), jnp.float32),
    mesh=plsc.VectorSubcoreMesh(
        core_axis_name="c", subcore_axis_name="s", num_cores=1
    ),
    scratch_types=[
        pltpu.VMEM((CHUNK,), jnp.float32),
        pltpu.VMEM((JROWS, 128), jnp.int32),
        pltpu.VMEM((S,), jnp.float32),
        pltpu.VMEM((NTILES, SEG_T), jnp.float32),
        pltpu.VMEM((SEG_T,), jnp.float32),
        pltpu.VMEM((SEG_T,), jnp.float32),
        pltpu.VMEM_SHARED((NTILES * S,), jnp.float32),
        pltpu.SemaphoreType.DMA,
    ],
)
def _segsum(
    v_hbm, b_hbm, shift_hbm, out_hbm,
    vals_v, idx_v, acc_v, red_v, shi_v, out_v, acc_sh, sem,
):
    _segsum_body(
        v_hbm, b_hbm, shift_hbm, out_hbm,
        vals_v, idx_v, acc_v, red_v, shi_v, out_v, acc_sh, sem,
    )


def kernel(h, batch, W0, b0, W1, b1, energy_scaling_coeff, energy_shifting_coeff):
    v = _node_scalars(h, W0, b0, W1, b1, energy_scaling_coeff).reshape(-1)
    pad = NPAD - N
    vp = jnp.concatenate([v, jnp.zeros((pad,), jnp.float32)])
    bp = jnp.concatenate([batch, jnp.zeros((pad,), jnp.int32)])
    bp = bp.reshape(NTILES, JROWS, 128)
    bp = bp + (jnp.arange(NTILES, dtype=jnp.int32) * S)[:, None, None]
    shift = jnp.broadcast_to(
        energy_shifting_coeff.astype(jnp.float32), (S,)
    )
    return _segsum(vp, bp, shift)
